# baseline (speedup 1.0000x reference)
import functools

import jax
import jax.numpy as jnp
from jax import lax
from jax.experimental import pallas as pl
from jax.experimental.pallas import tpu as pltpu
from jax.experimental.pallas import tpu_sc as plsc

_B = 16384
_F = 26
_D = 16
_N = _B * _F

_INFO = plsc.get_sparse_core_info()
_NC = _INFO.num_cores
_NS = _INFO.num_subcores
_NW = _NC * _NS
_PER_W = _N // _NW
_CHUNK = 832
_NCHUNK = _PER_W // _CHUNK
_LANES = 16


def _sc_kernel(idx_hbm, emb_hbm, mi_hbm, ms_hbm, mj_hbm,
               o0_hbm, o1_hbm, o2_hbm,
               idx_v, rows_v, wi_v, ws_v, wj_v,
               o1_v, o2_v, sem):
    wid = lax.axis_index("s") * _NC + lax.axis_index("c")
    base = wid * _PER_W

    def chunk_body(c, carry):
        off = base + c * _CHUNK
        pltpu.sync_copy(idx_hbm.at[pl.ds(off, _CHUNK)], idx_v)
        cp_r = pltpu.async_copy(emb_hbm.at[idx_v], rows_v, sem)
        cp_i = pltpu.async_copy(mi_hbm.at[idx_v], wi_v, sem)
        cp_s = pltpu.async_copy(ms_hbm.at[idx_v], ws_v, sem)
        cp_j = pltpu.async_copy(mj_hbm.at[idx_v], wj_v, sem)
        cp_r.wait()
        cp_i.wait()
        cp_s.wait()
        cp_j.wait()

        def blk_body(j, carry2):
            base_r = pl.multiple_of(j * _LANES, _LANES)
            sl = pl.ds(base_r, _LANES)
            wi = wi_v[sl]
            ws = ws_v[sl]
            wj = wj_v[sl]
            s_i = 2.0 / (1.0 + jnp.exp(-wi))
            s_s = 2.0 / (1.0 + jnp.exp(-ws))
            s_j = 2.0 / (1.0 + jnp.exp(-wj))
            gt = s_s > 1.0
            m1 = jnp.where(gt, s_s, s_i)
            m2 = jnp.where(gt, s_s, s_j)
            for l in range(_LANES):
                r = base_r + l
                e = rows_v[r, :]
                rows_v[r, :] = e * s_s[l]
                o1_v[r, :] = e * m1[l]
                o2_v[r, :] = e * m2[l]
            return carry2

        lax.fori_loop(0, _CHUNK // _LANES, blk_body, 0, unroll=False)

        pltpu.sync_copy(rows_v, o0_hbm.at[pl.ds(off, _CHUNK)])
        pltpu.sync_copy(o1_v, o1_hbm.at[pl.ds(off, _CHUNK)])
        pltpu.sync_copy(o2_v, o2_hbm.at[pl.ds(off, _CHUNK)])
        return carry

    lax.fori_loop(0, _NCHUNK, chunk_body, 0, unroll=False)


@jax.jit
def _run(idx, embedding, mi, ms, mj):
    out_sd = jax.ShapeDtypeStruct((_N, _D), jnp.float32)
    f = pl.kernel(
        _sc_kernel,
        out_type=(out_sd, out_sd, out_sd),
        mesh=plsc.VectorSubcoreMesh(core_axis_name="c", subcore_axis_name="s"),
        compiler_params=pltpu.CompilerParams(use_tc_tiling_on_sc=False),
        scratch_types=[
            pltpu.VMEM((_CHUNK,), jnp.int32),
            pltpu.VMEM((_CHUNK, _D), jnp.float32),
            pltpu.VMEM((_CHUNK,), jnp.float32),
            pltpu.VMEM((_CHUNK,), jnp.float32),
            pltpu.VMEM((_CHUNK,), jnp.float32),
            pltpu.VMEM((_CHUNK, _D), jnp.float32),
            pltpu.VMEM((_CHUNK, _D), jnp.float32),
            pltpu.SemaphoreType.DMA,
        ],
    )
    return f(idx, embedding, mi, ms, mj)


def kernel(x, embedding, mask_weight_i, mask_weight_s, mask_weight_j):
    idx = x.reshape(-1)
    mi = mask_weight_i.reshape(-1)
    ms = mask_weight_s.reshape(-1)
    mj = mask_weight_j.reshape(-1)
    o0, o1, o2 = _run(idx, embedding, mi, ms, mj)
    shape = (_B, _F, _D)
    return (o0.reshape(shape), o1.reshape(shape), o2.reshape(shape))


# speedup vs baseline: 1.0632x; 1.0632x over previous
import functools

import jax
import jax.numpy as jnp
from jax import lax
from jax.experimental import pallas as pl
from jax.experimental.pallas import tpu as pltpu
from jax.experimental.pallas import tpu_sc as plsc

_B = 16384
_F = 26
_D = 16
_N = _B * _F

_INFO = plsc.get_sparse_core_info()
_NC = _INFO.num_cores
_NS = _INFO.num_subcores
_NW = _NC * _NS
_PER_W = _N // _NW
_CHUNK = 832
_NCHUNK = _PER_W // _CHUNK
_NPAIR = _NCHUNK // 2
_LANES = 16


def _sc_kernel(idx_hbm, emb_hbm, mi_hbm, ms_hbm, mj_hbm,
               o0_hbm, o1_hbm, o2_hbm,
               idx0, rows0, wi0, ws0, wj0,
               idx1, rows1, wi1, ws1, wj1,
               oa0, oa1, oa2, ob0, ob1, ob2,
               sem_g0, sem_g1, sem_w0, sem_w1):
    wid = lax.axis_index("s") * _NC + lax.axis_index("c")
    base = wid * _PER_W

    g0 = (idx0, rows0, wi0, ws0, wj0, sem_g0)
    g1 = (idx1, rows1, wi1, ws1, wj1, sem_g1)
    o0 = (oa0, oa1, oa2, sem_w0)
    o1 = (ob0, ob1, ob2, sem_w1)

    def issue(c, g):
        idx_v, rows_v, wi_v, ws_v, wj_v, sem = g
        off = base + c * _CHUNK
        pltpu.sync_copy(idx_hbm.at[pl.ds(off, _CHUNK)], idx_v)
        pltpu.async_copy(emb_hbm.at[idx_v], rows_v, sem)
        pltpu.async_copy(mi_hbm.at[idx_v], wi_v, sem)
        pltpu.async_copy(ms_hbm.at[idx_v], ws_v, sem)
        pltpu.async_copy(mj_hbm.at[idx_v], wj_v, sem)

    def wait_gathers(g):
        idx_v, rows_v, wi_v, ws_v, wj_v, sem = g
        pltpu.make_async_copy(emb_hbm.at[idx_v], rows_v, sem).wait()
        pltpu.make_async_copy(mi_hbm.at[idx_v], wi_v, sem).wait()
        pltpu.make_async_copy(ms_hbm.at[idx_v], ws_v, sem).wait()
        pltpu.make_async_copy(mj_hbm.at[idx_v], wj_v, sem).wait()

    def fire_writes(c, o):
        e0, e1, e2, sem = o
        off = base + c * _CHUNK
        pltpu.async_copy(e0, o0_hbm.at[pl.ds(off, _CHUNK)], sem)
        pltpu.async_copy(e1, o1_hbm.at[pl.ds(off, _CHUNK)], sem)
        pltpu.async_copy(e2, o2_hbm.at[pl.ds(off, _CHUNK)], sem)

    def wait_writes(o):
        e0, e1, e2, sem = o
        pltpu.make_async_copy(e0, o0_hbm.at[pl.ds(base, _CHUNK)], sem).wait()
        pltpu.make_async_copy(e1, o1_hbm.at[pl.ds(base, _CHUNK)], sem).wait()
        pltpu.make_async_copy(e2, o2_hbm.at[pl.ds(base, _CHUNK)], sem).wait()

    def compute(g, o):
        idx_v, rows_v, wi_v, ws_v, wj_v, _ = g
        e0, e1, e2, _ = o

        def blk_body(j, carry2):
            base_r = pl.multiple_of(j * _LANES, _LANES)
            sl = pl.ds(base_r, _LANES)
            wi = wi_v[sl]
            ws = ws_v[sl]
            wj = wj_v[sl]
            s_i = 2.0 / (1.0 + jnp.exp(-wi))
            s_s = 2.0 / (1.0 + jnp.exp(-ws))
            s_j = 2.0 / (1.0 + jnp.exp(-wj))
            gt = s_s > 1.0
            m1 = jnp.where(gt, s_s, s_i)
            m2 = jnp.where(gt, s_s, s_j)
            for l in range(_LANES):
                r = base_r + l
                e = rows_v[r, :]
                e0[r, :] = e * s_s[l]
                e1[r, :] = e * m1[l]
                e2[r, :] = e * m2[l]
            return carry2

        lax.fori_loop(0, _CHUNK // _LANES, blk_body, 0, unroll=False)

    issue(0, g0)

    def pair_body(i, carry):
        c0 = i * 2
        c1 = c0 + 1
        issue(c1, g1)
        @pl.when(i >= 1)
        def _():
            wait_writes(o0)
        wait_gathers(g0)
        compute(g0, o0)
        fire_writes(c0, o0)
        @pl.when(i + 1 < _NPAIR)
        def _():
            issue(c0 + 2, g0)
        @pl.when(i >= 1)
        def _():
            wait_writes(o1)
        wait_gathers(g1)
        compute(g1, o1)
        fire_writes(c1, o1)
        return carry

    lax.fori_loop(0, _NPAIR, pair_body, 0, unroll=False)
    wait_writes(o0)
    wait_writes(o1)


@jax.jit
def _run(idx, embedding, mi, ms, mj):
    out_sd = jax.ShapeDtypeStruct((_N, _D), jnp.float32)
    gset = [
        pltpu.VMEM((_CHUNK,), jnp.int32),
        pltpu.VMEM((_CHUNK, _D), jnp.float32),
        pltpu.VMEM((_CHUNK,), jnp.float32),
        pltpu.VMEM((_CHUNK,), jnp.float32),
        pltpu.VMEM((_CHUNK,), jnp.float32),
    ]
    oset = [pltpu.VMEM((_CHUNK, _D), jnp.float32)] * 3
    f = pl.kernel(
        _sc_kernel,
        out_type=(out_sd, out_sd, out_sd),
        mesh=plsc.VectorSubcoreMesh(core_axis_name="c", subcore_axis_name="s"),
        compiler_params=pltpu.CompilerParams(use_tc_tiling_on_sc=False),
        scratch_types=gset + gset + oset + oset + [
            pltpu.SemaphoreType.DMA,
            pltpu.SemaphoreType.DMA,
            pltpu.SemaphoreType.DMA,
            pltpu.SemaphoreType.DMA,
        ],
    )
    return f(idx, embedding, mi, ms, mj)


def kernel(x, embedding, mask_weight_i, mask_weight_s, mask_weight_j):
    idx = x.reshape(-1)
    mi = mask_weight_i.reshape(-1)
    ms = mask_weight_s.reshape(-1)
    mj = mask_weight_j.reshape(-1)
    o0, o1, o2 = _run(idx, embedding, mi, ms, mj)
    shape = (_B, _F, _D)
    return (o0.reshape(shape), o1.reshape(shape), o2.reshape(shape))
